# baseline (speedup 1.0000x reference)
import functools

import jax
import jax.numpy as jnp
from jax import lax
from jax.experimental import pallas as pl
from jax.experimental.pallas import tpu as pltpu
from jax.experimental.pallas import tpu_sc as plsc

N = 10000
NP = 10112
ROWS = NP // 16
E = 160000
EP = 163840
H = 128
B = 8
OUT = 64
C = 128
NW = 32
CHUNKS = EP // (C * NW)
BLK = 1000
GRID = N // BLK

def _rsqrt_nr(d2):
    bits = plsc.bitcast(d2, jnp.int32)
    y = plsc.bitcast(jnp.int32(0x5F3759DF) - lax.shift_right_logical(bits, 1),
                     jnp.float32)
    for _ in range(4):
        y = y * (1.5 - ((0.5 * d2) * y) * y)
    return y


def _sc_prep_body(posf_hbm, rowp_hbm, colp_hbm, ea_out, posbuf, rbuf,
                  cbuf, eaout):
    cid = lax.axis_index("c")
    sid = lax.axis_index("s")
    w = cid * 16 + sid
    pltpu.sync_copy(posf_hbm, posbuf)

    def chunk(c, carry):
        base = pl.multiple_of((w * CHUNKS + c) * C, C)
        pltpu.sync_copy(rowp_hbm.at[pl.ds(base, C)], rbuf)
        pltpu.sync_copy(colp_hbm.at[pl.ds(base, C)], cbuf)

        def group(g, carry2):
            rv = rbuf[pl.ds(g * 16, 16)]
            cv = cbuf[pl.ds(g * 16, 16)]
            pr = [plsc.load_gather(posbuf, [rv * 4 + t]) for t in range(3)]
            pc = [plsc.load_gather(posbuf, [cv * 4 + t]) for t in range(3)]
            dx, dy, dz = pr[0] - pc[0], pr[1] - pc[1], pr[2] - pc[2]
            d2 = dx * dx + dy * dy + dz * dz
            y = _rsqrt_nr(d2)
            dist = d2 * y
            inv = 1.0 / (dist + 1e-8)
            eaout[pl.ds(0 * C + g * 16, 16)] = dx * inv
            eaout[pl.ds(1 * C + g * 16, 16)] = dy * inv
            eaout[pl.ds(2 * C + g * 16, 16)] = dz * inv
            eaout[pl.ds(3 * C + g * 16, 16)] = dist
            return carry2

        lax.fori_loop(0, 8, group, 0)
        for t in range(4):
            pltpu.sync_copy(eaout.at[pl.ds(t * C, C)],
                            ea_out.at[t, pl.ds(base, C)])
        return carry

    lax.fori_loop(0, CHUNKS, chunk, 0)


def _sc_edge_body(p_hbm, q_hbm, colp_hbm, rowp_hbm, r_hbm, eidx_hbm,
                  zer_hbm, s_out, dstidx, srcidx, ridx, pbuf, stab,
                  sem1, sem2, sem3):
    cid = lax.axis_index("c")
    sid = lax.axis_index("s")
    w = cid * 16 + sid
    pltpu.sync_copy(zer_hbm, stab.at[pl.ds(sid * ROWS, ROWS), :])
    plsc.subcore_barrier()

    def chunk(c, carry):
        base = pl.multiple_of((w * CHUNKS + c) * C, C)
        pltpu.sync_copy(colp_hbm.at[pl.ds(base, C)], dstidx)
        pltpu.sync_copy(rowp_hbm.at[pl.ds(base, C)], srcidx)
        pltpu.sync_copy(eidx_hbm.at[pl.ds(base, C)], ridx)
        cp1 = pltpu.async_copy(p_hbm.at[dstidx], pbuf, sem1)
        cp1.wait()
        cp3 = pltpu.async_copy(r_hbm.at[ridx], pbuf, sem3, add=True)
        cp2 = pltpu.async_copy(q_hbm.at[srcidx], pbuf, sem2, add=True)
        cp3.wait()
        cp2.wait()

        def edge(e, carry2):
            for j in range(8):
                sl = pl.ds(j * 16, 16)
                pbuf[e, sl] = jnp.maximum(pbuf[e, sl], 0.0)
            return carry2

        lax.fori_loop(0, C, edge, 0)
        pltpu.sync_copy(pbuf, stab.at[dstidx], add=True)
        return carry

    lax.fori_loop(0, CHUNKS, chunk, 0)
    plsc.subcore_barrier()
    pltpu.sync_copy(stab.at[pl.ds(sid * ROWS, ROWS), :],
                    s_out.at[cid, pl.ds(sid * ROWS, ROWS), :])


def _sc_deg_body(colp_hbm, one_hbm, zer_hbm, s_out, dstidx, onebuf, stab):
    cid = lax.axis_index("c")
    sid = lax.axis_index("s")
    w = cid * 16 + sid
    pltpu.sync_copy(one_hbm, onebuf)
    pltpu.sync_copy(zer_hbm, stab.at[pl.ds(sid * ROWS, ROWS), :])
    plsc.subcore_barrier()

    def chunk(c, carry):
        base = pl.multiple_of((w * CHUNKS + c) * C, C)
        pltpu.sync_copy(colp_hbm.at[pl.ds(base, C)], dstidx)
        pltpu.sync_copy(onebuf, stab.at[dstidx], add=True)
        return carry

    lax.fori_loop(0, CHUNKS, chunk, 0)
    plsc.subcore_barrier()
    pltpu.sync_copy(stab.at[pl.ds(sid * ROWS, ROWS), :],
                    s_out.at[cid, pl.ds(sid * ROWS, ROWS), :])


@functools.lru_cache(maxsize=1)
def _sc_kernels():
    mesh = plsc.VectorSubcoreMesh(core_axis_name="c", subcore_axis_name="s")
    params = pltpu.CompilerParams(needs_layout_passes=False)
    prep = pl.kernel(
        _sc_prep_body,
        out_type=jax.ShapeDtypeStruct((4, EP), jnp.float32),
        mesh=mesh,
        scratch_types=[pltpu.VMEM((NP * 4,), jnp.float32),
                       pltpu.VMEM((C,), jnp.int32),
                       pltpu.VMEM((C,), jnp.int32),
                       pltpu.VMEM((4 * C,), jnp.float32)],
        compiler_params=params,
    )
    edge = pl.kernel(
        _sc_edge_body,
        out_type=jax.ShapeDtypeStruct((2, NP, H), jnp.float32),
        mesh=mesh,
        scratch_types=[pltpu.VMEM((C,), jnp.int32),
                       pltpu.VMEM((C,), jnp.int32),
                       pltpu.VMEM((C,), jnp.int32),
                       pltpu.VMEM((C, H), jnp.float32),
                       pltpu.VMEM_SHARED((NP, H), jnp.float32),
                       pltpu.SemaphoreType.DMA,
                       pltpu.SemaphoreType.DMA,
                       pltpu.SemaphoreType.DMA],
        compiler_params=params,
    )
    deg = pl.kernel(
        _sc_deg_body,
        out_type=jax.ShapeDtypeStruct((2, NP, H), jnp.float32),
        mesh=mesh,
        scratch_types=[pltpu.VMEM((C,), jnp.int32),
                       pltpu.VMEM((C, H), jnp.float32),
                       pltpu.VMEM_SHARED((NP, H), jnp.float32)],
        compiler_params=params,
    )
    return prep, edge, deg


def _dot(a, b):
    return jnp.dot(a, b, preferred_element_type=jnp.float32)


BLKE = 2048


def _tc_r_body(ea_ref, w_ref, r0_ref, r1_ref, r2_ref, r3_ref):
    r = _dot(ea_ref[...], w_ref[...])
    r0_ref[...] = r[:, 0 * H:1 * H]
    r1_ref[...] = r[:, 1 * H:2 * H]
    r2_ref[...] = r[:, 2 * H:3 * H]
    r3_ref[...] = r[:, 3 * H:4 * H]


def _tc_r(ea_t, w_all):
    espec = pl.BlockSpec((BLKE, 4), lambda i: (i, 0))
    rspec = pl.BlockSpec((BLKE, H), lambda i: (i, 0))
    shp = jax.ShapeDtypeStruct((EP, H), jnp.float32)
    return pl.pallas_call(
        _tc_r_body,
        grid=(EP // BLKE,),
        in_specs=[espec, pl.BlockSpec((4, 4 * H), lambda i: (0, 0))],
        out_specs=[rspec, rspec, rspec, rspec],
        out_shape=[shp, shp, shp, shp],
    )(ea_t, w_all)


def _tc_in_pq_body(x_ref, win_ref, bin_ref, w1d_ref, w1j_ref, mb1_ref,
                   h_ref, p_ref, q_ref, ss_ref):
    h = jnp.maximum(_dot(x_ref[...], win_ref[...]) + bin_ref[...], 0.0)
    h_ref[...] = h
    p = _dot(h, w1d_ref[...]) + mb1_ref[...]
    q = _dot(h, w1j_ref[...])
    p_ref[...] = p
    q_ref[...] = q
    ss_ref[...] = jnp.maximum(p + q, 0.0)


def _tc_in_pq(x, Win, binr, W1d, W1j, mb1r):
    wspec = pl.BlockSpec((H, H), lambda i: (0, 0))
    bspec = pl.BlockSpec((1, H), lambda i: (0, 0))
    nspec = pl.BlockSpec((BLK, H), lambda i: (i, 0))
    return pl.pallas_call(
        _tc_in_pq_body,
        grid=(GRID,),
        in_specs=[nspec, wspec, bspec, wspec, wspec, bspec],
        out_specs=[nspec, nspec, nspec, nspec],
        out_shape=[jax.ShapeDtypeStruct((N, H), jnp.float32),
                   jax.ShapeDtypeStruct((NP, H), jnp.float32),
                   jax.ShapeDtypeStruct((NP, H), jnp.float32),
                   jax.ShapeDtypeStruct((N, H), jnp.float32)],
    )(x, Win, binr, W1d, W1j, mb1r)


def _ln_update(h_ref, s_ref, ss_ref, deg_ref, mW2_ref, mb2_ref, ua_ref,
               ub_ref, ub1_ref, uW2_ref, ub2_ref, g_ref, bt_ref):
    s = s_ref[0] + s_ref[1] + ss_ref[...]
    deg = deg_ref[0, :, 0] + deg_ref[1, :, 0] + 1.0
    agg = _dot(s, mW2_ref[...]) + deg[:, None] * mb2_ref[...]
    u = jnp.maximum(_dot(h_ref[...], ua_ref[...]) + _dot(agg, ub_ref[...])
                    + ub1_ref[...], 0.0)
    u = _dot(u, uW2_ref[...]) + ub2_ref[...] + h_ref[...]
    mu = jnp.mean(u, axis=-1, keepdims=True)
    var = jnp.mean((u - mu) ** 2, axis=-1, keepdims=True)
    return (u - mu) / jnp.sqrt(var + 1e-5) * g_ref[...] + bt_ref[...]


def _tc_upd_body(h_ref, s_ref, ss_ref, deg_ref, mW2_ref, mb2_ref, ua_ref,
                 ub_ref, ub1_ref, uW2_ref, ub2_ref, g_ref, bt_ref,
                 w1d_ref, w1j_ref, mb1_ref, h_out, p_out, q_out, ss_out):
    hn = _ln_update(h_ref, s_ref, ss_ref, deg_ref, mW2_ref, mb2_ref, ua_ref,
                    ub_ref, ub1_ref, uW2_ref, ub2_ref, g_ref, bt_ref)
    h_out[...] = hn
    p = _dot(hn, w1d_ref[...]) + mb1_ref[...]
    q = _dot(hn, w1j_ref[...])
    p_out[...] = p
    q_out[...] = q
    ss_out[...] = jnp.maximum(p + q, 0.0)


def _tc_upd_last_body(h_ref, s_ref, ss_ref, deg_ref, mW2_ref, mb2_ref,
                      ua_ref, ub_ref, ub1_ref, uW2_ref, ub2_ref, g_ref,
                      bt_ref, h_out):
    h_out[...] = _ln_update(h_ref, s_ref, ss_ref, deg_ref, mW2_ref, mb2_ref,
                            ua_ref, ub_ref, ub1_ref, uW2_ref, ub2_ref,
                            g_ref, bt_ref)


def _tc_update(h, S, SS, degp, mW2, mb2r, ua, ub, ub1r, uW2, ub2r, gr, btr,
               nxt):
    wspec = pl.BlockSpec((H, H), lambda i: (0, 0))
    bspec = pl.BlockSpec((1, H), lambda i: (0, 0))
    nspec = pl.BlockSpec((BLK, H), lambda i: (i, 0))
    sspec = pl.BlockSpec((2, BLK, H), lambda i: (0, i, 0))
    dspec = pl.BlockSpec((2, BLK, H), lambda i: (0, i, 0))
    base_in = [nspec, sspec, nspec, dspec, wspec, bspec, wspec, wspec,
               bspec, wspec, bspec, bspec, bspec]
    args = [h, S, SS, degp, mW2, mb2r, ua, ub, ub1r, uW2, ub2r, gr, btr]
    if nxt is None:
        return pl.pallas_call(
            _tc_upd_last_body,
            grid=(GRID,),
            in_specs=base_in,
            out_specs=[nspec],
            out_shape=[jax.ShapeDtypeStruct((N, H), jnp.float32)],
        )(*args)
    W1d, W1j, mb1r = nxt
    return pl.pallas_call(
        _tc_upd_body,
        grid=(GRID,),
        in_specs=base_in + [wspec, wspec, bspec],
        out_specs=[nspec, nspec, nspec, nspec],
        out_shape=[jax.ShapeDtypeStruct((N, H), jnp.float32),
                   jax.ShapeDtypeStruct((NP, H), jnp.float32),
                   jax.ShapeDtypeStruct((NP, H), jnp.float32),
                   jax.ShapeDtypeStruct((N, H), jnp.float32)],
    )(*args, W1d, W1j, mb1r)


def _tc_pool_body(h_ref, b_ref, ow1_ref, ob1_ref, ow2_ref, ob2_ref, z_ref,
                  acc, cnt):
    i = pl.program_id(0)

    @pl.when(i == 0)
    def _():
        acc[...] = jnp.zeros_like(acc)
        cnt[...] = jnp.zeros_like(cnt)

    oh = (b_ref[...] == lax.broadcasted_iota(jnp.int32, (BLK, B), 1)
          ).astype(jnp.float32)
    acc[...] += lax.dot_general(oh, h_ref[...], (((0,), (0,)), ((), ())),
                                preferred_element_type=jnp.float32)
    cnt[...] += jnp.broadcast_to(jnp.sum(oh, axis=0)[:, None], (B, H))

    @pl.when(i == GRID - 1)
    def _():
        pooled = acc[...] / jnp.maximum(cnt[...], 1.0)
        t = jnp.maximum(_dot(pooled, ow1_ref[...]) + ob1_ref[...], 0.0)
        z_ref[...] = _dot(t, ow2_ref[...]) + ob2_ref[...]


def _tc_pool(h, batch2d, oW1, ob1r, oW2, ob2r):
    return pl.pallas_call(
        _tc_pool_body,
        grid=(GRID,),
        in_specs=[pl.BlockSpec((BLK, H), lambda i: (i, 0)),
                  pl.BlockSpec((BLK, 1), lambda i: (i, 0)),
                  pl.BlockSpec((H, OUT), lambda i: (0, 0)),
                  pl.BlockSpec((1, OUT), lambda i: (0, 0)),
                  pl.BlockSpec((OUT, OUT), lambda i: (0, 0)),
                  pl.BlockSpec((1, OUT), lambda i: (0, 0))],
        out_specs=[pl.BlockSpec((B, OUT), lambda i: (0, 0))],
        out_shape=[jax.ShapeDtypeStruct((B, OUT), jnp.float32)],
        scratch_shapes=[pltpu.VMEM((B, H), jnp.float32),
                        pltpu.VMEM((B, H), jnp.float32)],
    )(h, batch2d, oW1, ob1r, oW2, ob2r)[0]


def kernel(x, pos, edge_index, batch, Win, bin_, layers, oW1, ob1, oW2, ob2):
    f32 = jnp.float32
    row = edge_index[0]
    col = edge_index[1]
    pad = EP - E
    rowp = jnp.concatenate([row, jnp.full((pad,), N, jnp.int32)])
    colp = jnp.concatenate([col, jnp.full((pad,), NP - 1, jnp.int32)])
    posf = jnp.pad(pos, ((0, NP - N), (0, 1))).reshape(-1)
    zer128 = jnp.zeros((ROWS, H), f32)
    one128 = jnp.ones((C, H), f32)
    eidx = jnp.arange(EP, dtype=jnp.int32)

    sc_prep, sc_edge, sc_deg = _sc_kernels()
    ea = sc_prep(posf, rowp, colp)
    w_all = jnp.concatenate([layers[l][0][2 * H:] for l in range(4)],
                            axis=1).astype(f32)
    R = _tc_r(jnp.transpose(ea), w_all)
    degp = sc_deg(colp, one128, zer128)

    (mW1, mb1, *_rest) = layers[0]
    h, P, Q, SS = _tc_in_pq(x, Win, bin_.reshape(1, H),
                            mW1[:H], mW1[H:2 * H], mb1.reshape(1, H))
    for l in range(4):
        mW1, mb1, mW2, mb2, uW1, ub1, uW2, ub2, g, bt = layers[l]
        S = sc_edge(P, Q, colp, rowp, R[l], eidx, zer128)
        if l < 3:
            nW1, nb1 = layers[l + 1][0], layers[l + 1][1]
            nxt = (nW1[:H], nW1[H:2 * H], nb1.reshape(1, H))
        else:
            nxt = None
        res = _tc_update(h, S, SS, degp, mW2, mb2.reshape(1, H),
                         uW1[:H], uW1[H:], ub1.reshape(1, H), uW2,
                         ub2.reshape(1, H), g.reshape(1, H),
                         bt.reshape(1, H), nxt)
        if l < 3:
            h, P, Q, SS = res
        else:
            h = res[0]

    return _tc_pool(h, batch.reshape(N, 1), oW1, ob1.reshape(1, OUT),
                    oW2, ob2.reshape(1, OUT))


# speedup vs baseline: 1.5616x; 1.5616x over previous
import functools

import jax
import jax.numpy as jnp
from jax import lax
from jax.experimental import pallas as pl
from jax.experimental.pallas import tpu as pltpu
from jax.experimental.pallas import tpu_sc as plsc

N = 10000
NP = 10112
ROWS = NP // 16
E = 160000
EP = 163840
H = 128
B = 8
OUT = 64
C = 64
NW = 32
CHUNKS = EP // (C * NW)
BLK = 1000
GRID = N // BLK

def _rsqrt_nr(d2):
    bits = plsc.bitcast(d2, jnp.int32)
    y = plsc.bitcast(jnp.int32(0x5F3759DF) - lax.shift_right_logical(bits, 1),
                     jnp.float32)
    for _ in range(4):
        y = y * (1.5 - ((0.5 * d2) * y) * y)
    return y


def _sc_prep_body(posf_hbm, rowp_hbm, colp_hbm, ea_out, posbuf, rbuf,
                  cbuf, eaout):
    cid = lax.axis_index("c")
    sid = lax.axis_index("s")
    w = cid * 16 + sid
    pltpu.sync_copy(posf_hbm, posbuf)

    def chunk(c, carry):
        base = pl.multiple_of((w * CHUNKS + c) * C, C)
        pltpu.sync_copy(rowp_hbm.at[pl.ds(base, C)], rbuf)
        pltpu.sync_copy(colp_hbm.at[pl.ds(base, C)], cbuf)

        def group(g, carry2):
            rv = rbuf[pl.ds(g * 16, 16)]
            cv = cbuf[pl.ds(g * 16, 16)]
            pr = [plsc.load_gather(posbuf, [rv * 4 + t]) for t in range(3)]
            pc = [plsc.load_gather(posbuf, [cv * 4 + t]) for t in range(3)]
            dx, dy, dz = pr[0] - pc[0], pr[1] - pc[1], pr[2] - pc[2]
            d2 = dx * dx + dy * dy + dz * dz
            y = _rsqrt_nr(d2)
            dist = d2 * y
            inv = 1.0 / (dist + 1e-8)
            eaout[pl.ds(0 * C + g * 16, 16)] = dx * inv
            eaout[pl.ds(1 * C + g * 16, 16)] = dy * inv
            eaout[pl.ds(2 * C + g * 16, 16)] = dz * inv
            eaout[pl.ds(3 * C + g * 16, 16)] = dist
            return carry2

        lax.fori_loop(0, C // 16, group, 0)
        for t in range(4):
            pltpu.sync_copy(eaout.at[pl.ds(t * C, C)],
                            ea_out.at[t, pl.ds(base, C)])
        return carry

    lax.fori_loop(0, CHUNKS, chunk, 0)


def _sc_edge_body(p_hbm, q_hbm, colp_hbm, rowp_hbm, r_hbm, zer_hbm,
                  s_out, dstidx0, srcidx0, pbuf0, qbuf0, rbuf0,
                  dstidx1, srcidx1, pbuf1, qbuf1, rbuf1, stab,
                  sp0, sq0, sr0, ss0, sp1, sq1, sr1, ss1):
    cid = lax.axis_index("c")
    sid = lax.axis_index("s")
    w = cid * 16 + sid
    pltpu.sync_copy(zer_hbm, stab.at[pl.ds(sid * ROWS, ROWS), :])
    plsc.subcore_barrier()

    sets = ((dstidx0, srcidx0, pbuf0, qbuf0, rbuf0, sp0, sq0, sr0, ss0),
            (dstidx1, srcidx1, pbuf1, qbuf1, rbuf1, sp1, sq1, sr1, ss1))

    def issue(c, st):
        dstidx, srcidx, pbuf, qbuf, rbuf, sp, sq, sr, _ = st
        base = pl.multiple_of((w * CHUNKS + c) * C, C)
        pltpu.sync_copy(colp_hbm.at[pl.ds(base, C)], dstidx)
        pltpu.sync_copy(rowp_hbm.at[pl.ds(base, C)], srcidx)
        cpr = pltpu.async_copy(r_hbm.at[pl.ds(base, C)], rbuf, sr)
        cpp = pltpu.async_copy(p_hbm.at[dstidx], pbuf, sp)
        cpq = pltpu.async_copy(q_hbm.at[srcidx], qbuf, sq)
        return (cpp, cpq, cpr)

    def compute(st):
        dstidx, srcidx, pbuf, qbuf, rbuf, _, _, _, ss = st

        def edge(e, carry2):
            for j in range(8):
                sl = pl.ds(j * 16, 16)
                pbuf[e, sl] = jnp.maximum(
                    pbuf[e, sl] + qbuf[e, sl] + rbuf[e, sl], 0.0)
            return carry2

        lax.fori_loop(0, C, edge, 0)
        return pltpu.async_copy(pbuf, stab.at[dstidx], ss, add=True)

    gat = [None, None]
    sca = [None, None]
    for c in range(CHUNKS + 1):
        b = c % 2
        if c < CHUNKS:
            if sca[b] is not None:
                sca[b].wait()
            gat[b] = issue(c, sets[b])
        if c >= 1:
            b1 = (c - 1) % 2
            for d in gat[b1]:
                d.wait()
            sca[b1] = compute(sets[b1])
    sca[(CHUNKS - 1) % 2].wait()
    if CHUNKS >= 2:
        sca[CHUNKS % 2].wait()
    plsc.subcore_barrier()
    pltpu.sync_copy(stab.at[pl.ds(sid * ROWS, ROWS), :],
                    s_out.at[cid, pl.ds(sid * ROWS, ROWS), :])


def _sc_deg_body(colp_hbm, one_hbm, zer_hbm, s_out, dstidx, onebuf, stab):
    cid = lax.axis_index("c")
    sid = lax.axis_index("s")
    w = cid * 16 + sid
    pltpu.sync_copy(one_hbm, onebuf)
    pltpu.sync_copy(zer_hbm, stab.at[pl.ds(sid * ROWS, ROWS), :])
    plsc.subcore_barrier()

    def chunk(c, carry):
        base = pl.multiple_of((w * CHUNKS + c) * C, C)
        pltpu.sync_copy(colp_hbm.at[pl.ds(base, C)], dstidx)
        pltpu.sync_copy(onebuf, stab.at[dstidx], add=True)
        return carry

    lax.fori_loop(0, CHUNKS, chunk, 0)
    plsc.subcore_barrier()
    pltpu.sync_copy(stab.at[pl.ds(sid * ROWS, ROWS), :],
                    s_out.at[cid, pl.ds(sid * ROWS, ROWS), :])


@functools.lru_cache(maxsize=1)
def _sc_kernels():
    mesh = plsc.VectorSubcoreMesh(core_axis_name="c", subcore_axis_name="s")
    params = pltpu.CompilerParams(needs_layout_passes=False)
    prep = pl.kernel(
        _sc_prep_body,
        out_type=jax.ShapeDtypeStruct((4, EP), jnp.float32),
        mesh=mesh,
        scratch_types=[pltpu.VMEM((NP * 4,), jnp.float32),
                       pltpu.VMEM((C,), jnp.int32),
                       pltpu.VMEM((C,), jnp.int32),
                       pltpu.VMEM((4 * C,), jnp.float32)],
        compiler_params=params,
    )
    edge = pl.kernel(
        _sc_edge_body,
        out_type=jax.ShapeDtypeStruct((2, NP, H), jnp.float32),
        mesh=mesh,
        scratch_types=[pltpu.VMEM((C,), jnp.int32),
                       pltpu.VMEM((C,), jnp.int32),
                       pltpu.VMEM((C, H), jnp.float32),
                       pltpu.VMEM((C, H), jnp.float32),
                       pltpu.VMEM((C, H), jnp.float32),
                       pltpu.VMEM((C,), jnp.int32),
                       pltpu.VMEM((C,), jnp.int32),
                       pltpu.VMEM((C, H), jnp.float32),
                       pltpu.VMEM((C, H), jnp.float32),
                       pltpu.VMEM((C, H), jnp.float32),
                       pltpu.VMEM_SHARED((NP, H), jnp.float32),
                       pltpu.SemaphoreType.DMA,
                       pltpu.SemaphoreType.DMA,
                       pltpu.SemaphoreType.DMA,
                       pltpu.SemaphoreType.DMA,
                       pltpu.SemaphoreType.DMA,
                       pltpu.SemaphoreType.DMA,
                       pltpu.SemaphoreType.DMA,
                       pltpu.SemaphoreType.DMA],
        compiler_params=params,
    )
    deg = pl.kernel(
        _sc_deg_body,
        out_type=jax.ShapeDtypeStruct((2, NP, H), jnp.float32),
        mesh=mesh,
        scratch_types=[pltpu.VMEM((C,), jnp.int32),
                       pltpu.VMEM((C, H), jnp.float32),
                       pltpu.VMEM_SHARED((NP, H), jnp.float32)],
        compiler_params=params,
    )
    return prep, edge, deg


def _dot(a, b):
    return jnp.dot(a, b, preferred_element_type=jnp.float32)


BLKE = 2048


def _tc_r_body(ea_ref, w_ref, r0_ref, r1_ref, r2_ref, r3_ref):
    r = _dot(ea_ref[...], w_ref[...])
    r0_ref[...] = r[:, 0 * H:1 * H]
    r1_ref[...] = r[:, 1 * H:2 * H]
    r2_ref[...] = r[:, 2 * H:3 * H]
    r3_ref[...] = r[:, 3 * H:4 * H]


def _tc_r(ea_t, w_all):
    espec = pl.BlockSpec((BLKE, 4), lambda i: (i, 0))
    rspec = pl.BlockSpec((BLKE, H), lambda i: (i, 0))
    shp = jax.ShapeDtypeStruct((EP, H), jnp.float32)
    return pl.pallas_call(
        _tc_r_body,
        grid=(EP // BLKE,),
        in_specs=[espec, pl.BlockSpec((4, 4 * H), lambda i: (0, 0))],
        out_specs=[rspec, rspec, rspec, rspec],
        out_shape=[shp, shp, shp, shp],
    )(ea_t, w_all)


def _tc_in_pq_body(x_ref, win_ref, bin_ref, w1d_ref, w1j_ref, mb1_ref,
                   h_ref, p_ref, q_ref, ss_ref):
    h = jnp.maximum(_dot(x_ref[...], win_ref[...]) + bin_ref[...], 0.0)
    h_ref[...] = h
    p = _dot(h, w1d_ref[...]) + mb1_ref[...]
    q = _dot(h, w1j_ref[...])
    p_ref[...] = p
    q_ref[...] = q
    ss_ref[...] = jnp.maximum(p + q, 0.0)


def _tc_in_pq(x, Win, binr, W1d, W1j, mb1r):
    wspec = pl.BlockSpec((H, H), lambda i: (0, 0))
    bspec = pl.BlockSpec((1, H), lambda i: (0, 0))
    nspec = pl.BlockSpec((BLK, H), lambda i: (i, 0))
    return pl.pallas_call(
        _tc_in_pq_body,
        grid=(GRID,),
        in_specs=[nspec, wspec, bspec, wspec, wspec, bspec],
        out_specs=[nspec, nspec, nspec, nspec],
        out_shape=[jax.ShapeDtypeStruct((N, H), jnp.float32),
                   jax.ShapeDtypeStruct((NP, H), jnp.float32),
                   jax.ShapeDtypeStruct((NP, H), jnp.float32),
                   jax.ShapeDtypeStruct((N, H), jnp.float32)],
    )(x, Win, binr, W1d, W1j, mb1r)


def _ln_update(h_ref, s_ref, ss_ref, deg_ref, mW2_ref, mb2_ref, ua_ref,
               ub_ref, ub1_ref, uW2_ref, ub2_ref, g_ref, bt_ref):
    s = s_ref[0] + s_ref[1] + ss_ref[...]
    deg = deg_ref[0, :, 0] + deg_ref[1, :, 0] + 1.0
    agg = _dot(s, mW2_ref[...]) + deg[:, None] * mb2_ref[...]
    u = jnp.maximum(_dot(h_ref[...], ua_ref[...]) + _dot(agg, ub_ref[...])
                    + ub1_ref[...], 0.0)
    u = _dot(u, uW2_ref[...]) + ub2_ref[...] + h_ref[...]
    mu = jnp.mean(u, axis=-1, keepdims=True)
    var = jnp.mean((u - mu) ** 2, axis=-1, keepdims=True)
    return (u - mu) / jnp.sqrt(var + 1e-5) * g_ref[...] + bt_ref[...]


def _tc_upd_body(h_ref, s_ref, ss_ref, deg_ref, mW2_ref, mb2_ref, ua_ref,
                 ub_ref, ub1_ref, uW2_ref, ub2_ref, g_ref, bt_ref,
                 w1d_ref, w1j_ref, mb1_ref, h_out, p_out, q_out, ss_out):
    hn = _ln_update(h_ref, s_ref, ss_ref, deg_ref, mW2_ref, mb2_ref, ua_ref,
                    ub_ref, ub1_ref, uW2_ref, ub2_ref, g_ref, bt_ref)
    h_out[...] = hn
    p = _dot(hn, w1d_ref[...]) + mb1_ref[...]
    q = _dot(hn, w1j_ref[...])
    p_out[...] = p
    q_out[...] = q
    ss_out[...] = jnp.maximum(p + q, 0.0)


def _tc_upd_last_body(h_ref, s_ref, ss_ref, deg_ref, mW2_ref, mb2_ref,
                      ua_ref, ub_ref, ub1_ref, uW2_ref, ub2_ref, g_ref,
                      bt_ref, h_out):
    h_out[...] = _ln_update(h_ref, s_ref, ss_ref, deg_ref, mW2_ref, mb2_ref,
                            ua_ref, ub_ref, ub1_ref, uW2_ref, ub2_ref,
                            g_ref, bt_ref)


def _tc_update(h, S, SS, degp, mW2, mb2r, ua, ub, ub1r, uW2, ub2r, gr, btr,
               nxt):
    wspec = pl.BlockSpec((H, H), lambda i: (0, 0))
    bspec = pl.BlockSpec((1, H), lambda i: (0, 0))
    nspec = pl.BlockSpec((BLK, H), lambda i: (i, 0))
    sspec = pl.BlockSpec((2, BLK, H), lambda i: (0, i, 0))
    dspec = pl.BlockSpec((2, BLK, H), lambda i: (0, i, 0))
    base_in = [nspec, sspec, nspec, dspec, wspec, bspec, wspec, wspec,
               bspec, wspec, bspec, bspec, bspec]
    args = [h, S, SS, degp, mW2, mb2r, ua, ub, ub1r, uW2, ub2r, gr, btr]
    if nxt is None:
        return pl.pallas_call(
            _tc_upd_last_body,
            grid=(GRID,),
            in_specs=base_in,
            out_specs=[nspec],
            out_shape=[jax.ShapeDtypeStruct((N, H), jnp.float32)],
        )(*args)
    W1d, W1j, mb1r = nxt
    return pl.pallas_call(
        _tc_upd_body,
        grid=(GRID,),
        in_specs=base_in + [wspec, wspec, bspec],
        out_specs=[nspec, nspec, nspec, nspec],
        out_shape=[jax.ShapeDtypeStruct((N, H), jnp.float32),
                   jax.ShapeDtypeStruct((NP, H), jnp.float32),
                   jax.ShapeDtypeStruct((NP, H), jnp.float32),
                   jax.ShapeDtypeStruct((N, H), jnp.float32)],
    )(*args, W1d, W1j, mb1r)


def _tc_pool_body(h_ref, b_ref, ow1_ref, ob1_ref, ow2_ref, ob2_ref, z_ref,
                  acc, cnt):
    i = pl.program_id(0)

    @pl.when(i == 0)
    def _():
        acc[...] = jnp.zeros_like(acc)
        cnt[...] = jnp.zeros_like(cnt)

    oh = (b_ref[...] == lax.broadcasted_iota(jnp.int32, (BLK, B), 1)
          ).astype(jnp.float32)
    acc[...] += lax.dot_general(oh, h_ref[...], (((0,), (0,)), ((), ())),
                                preferred_element_type=jnp.float32)
    cnt[...] += jnp.broadcast_to(jnp.sum(oh, axis=0)[:, None], (B, H))

    @pl.when(i == GRID - 1)
    def _():
        pooled = acc[...] / jnp.maximum(cnt[...], 1.0)
        t = jnp.maximum(_dot(pooled, ow1_ref[...]) + ob1_ref[...], 0.0)
        z_ref[...] = _dot(t, ow2_ref[...]) + ob2_ref[...]


def _tc_pool(h, batch2d, oW1, ob1r, oW2, ob2r):
    return pl.pallas_call(
        _tc_pool_body,
        grid=(GRID,),
        in_specs=[pl.BlockSpec((BLK, H), lambda i: (i, 0)),
                  pl.BlockSpec((BLK, 1), lambda i: (i, 0)),
                  pl.BlockSpec((H, OUT), lambda i: (0, 0)),
                  pl.BlockSpec((1, OUT), lambda i: (0, 0)),
                  pl.BlockSpec((OUT, OUT), lambda i: (0, 0)),
                  pl.BlockSpec((1, OUT), lambda i: (0, 0))],
        out_specs=[pl.BlockSpec((B, OUT), lambda i: (0, 0))],
        out_shape=[jax.ShapeDtypeStruct((B, OUT), jnp.float32)],
        scratch_shapes=[pltpu.VMEM((B, H), jnp.float32),
                        pltpu.VMEM((B, H), jnp.float32)],
    )(h, batch2d, oW1, ob1r, oW2, ob2r)[0]


def kernel(x, pos, edge_index, batch, Win, bin_, layers, oW1, ob1, oW2, ob2):
    f32 = jnp.float32
    row = edge_index[0]
    col = edge_index[1]
    pad = EP - E
    rowp = jnp.concatenate([row, jnp.full((pad,), N, jnp.int32)])
    colp = jnp.concatenate([col, jnp.full((pad,), NP - 1, jnp.int32)])
    posf = jnp.pad(pos, ((0, NP - N), (0, 1))).reshape(-1)
    zer128 = jnp.zeros((ROWS, H), f32)
    one128 = jnp.ones((C, H), f32)

    sc_prep, sc_edge, sc_deg = _sc_kernels()
    ea = sc_prep(posf, rowp, colp)
    w_all = jnp.concatenate([layers[l][0][2 * H:] for l in range(4)],
                            axis=1).astype(f32)
    R = _tc_r(jnp.transpose(ea), w_all)
    degp = sc_deg(colp, one128, zer128)

    (mW1, mb1, *_rest) = layers[0]
    h, P, Q, SS = _tc_in_pq(x, Win, bin_.reshape(1, H),
                            mW1[:H], mW1[H:2 * H], mb1.reshape(1, H))
    for l in range(4):
        mW1, mb1, mW2, mb2, uW1, ub1, uW2, ub2, g, bt = layers[l]
        S = sc_edge(P, Q, colp, rowp, R[l], zer128)
        if l < 3:
            nW1, nb1 = layers[l + 1][0], layers[l + 1][1]
            nxt = (nW1[:H], nW1[H:2 * H], nb1.reshape(1, H))
        else:
            nxt = None
        res = _tc_update(h, S, SS, degp, mW2, mb2.reshape(1, H),
                         uW1[:H], uW1[H:], ub1.reshape(1, H), uW2,
                         ub2.reshape(1, H), g.reshape(1, H),
                         bt.reshape(1, H), nxt)
        if l < 3:
            h, P, Q, SS = res
        else:
            h = res[0]

    return _tc_pool(h, batch.reshape(N, 1), oW1, ob1.reshape(1, OUT),
                    oW2, ob2.reshape(1, OUT))
